# speedup vs baseline: 2.9768x; 2.9768x over previous
import functools

import jax
import jax.numpy as jnp
from jax import lax
from jax.experimental import pallas as pl
from jax.experimental.pallas import tpu as pltpu
from jax.experimental.pallas import tpu_sc as plsc

NUM_EMBEDDINGS = 100000
EMBED_DIM = 128
BATCH = 4096
HIST = 50
TOTAL = BATCH * HIST

NUM_CORES = 2
NUM_SUBCORES = 16
NUM_WORKERS = NUM_CORES * NUM_SUBCORES
PER_WORKER = TOTAL // NUM_WORKERS
CHUNK = 128
NUM_CHUNKS = PER_WORKER // CHUNK

_mesh = plsc.VectorSubcoreMesh(
    core_axis_name="c",
    subcore_axis_name="s",
    num_cores=NUM_CORES,
    num_subcores=NUM_SUBCORES,
)


@functools.partial(
    pl.kernel,
    out_type=jax.ShapeDtypeStruct((TOTAL, EMBED_DIM), jnp.float32),
    mesh=_mesh,
    scratch_types=[
        pltpu.VMEM((NUM_CHUNKS, CHUNK), jnp.int32),
        pltpu.VMEM((CHUNK, EMBED_DIM), jnp.float32),
        pltpu.SemaphoreType.DMA,
    ],
)
def _gather_kernel(idx_hbm, table_hbm, out_hbm, idx_v, rows_v, sem):
    wid = lax.axis_index("s") * NUM_CORES + lax.axis_index("c")
    base = wid * PER_WORKER
    pltpu.sync_copy(idx_hbm.at[wid], idx_v)

    def step(j, _):
        pltpu.async_copy(table_hbm.at[idx_v.at[j]], rows_v, sem).wait()
        pltpu.sync_copy(rows_v, out_hbm.at[pl.ds(base + j * CHUNK, CHUNK)])
        return 0

    lax.fori_loop(0, NUM_CHUNKS, step, 0)


def kernel(token_ids, table):
    idx = token_ids.reshape(NUM_WORKERS, NUM_CHUNKS, CHUNK).astype(jnp.int32)
    out = _gather_kernel(idx, table)
    return out.reshape(BATCH, HIST, EMBED_DIM)


# speedup vs baseline: 3.3150x; 1.1136x over previous
import functools

import jax
import jax.numpy as jnp
from jax import lax
from jax.experimental import pallas as pl
from jax.experimental.pallas import tpu as pltpu
from jax.experimental.pallas import tpu_sc as plsc

NUM_EMBEDDINGS = 100000
EMBED_DIM = 128
BATCH = 4096
HIST = 50
TOTAL = BATCH * HIST

NUM_CORES = 2
NUM_SUBCORES = 16
NUM_WORKERS = NUM_CORES * NUM_SUBCORES
PER_WORKER = TOTAL // NUM_WORKERS
CHUNK = 128
NUM_CHUNKS = PER_WORKER // CHUNK
NBUF = 5
ROUNDS = NUM_CHUNKS // NBUF

_mesh = plsc.VectorSubcoreMesh(
    core_axis_name="c",
    subcore_axis_name="s",
    num_cores=NUM_CORES,
    num_subcores=NUM_SUBCORES,
)


@functools.partial(
    pl.kernel,
    out_type=jax.ShapeDtypeStruct((TOTAL, EMBED_DIM), jnp.float32),
    mesh=_mesh,
    scratch_types=[
        pltpu.VMEM((NUM_CHUNKS, CHUNK), jnp.int32),
        [pltpu.VMEM((CHUNK, EMBED_DIM), jnp.float32)] * NBUF,
        [pltpu.SemaphoreType.DMA] * NBUF,
        [pltpu.SemaphoreType.DMA] * NBUF,
    ],
)
def _gather_kernel(idx_hbm, table_hbm, out_hbm, idx_v, bufs, gsems, wsems):
    wid = lax.axis_index("s") * NUM_CORES + lax.axis_index("c")
    base = wid * PER_WORKER
    pltpu.sync_copy(idx_hbm.at[wid], idx_v)

    for b in range(NBUF):
        pltpu.async_copy(table_hbm.at[idx_v.at[b]], bufs[b], gsems[b])

    def round_body(r, _):
        for b in range(NBUF):
            j = r * NBUF + b
            pltpu.make_async_copy(table_hbm.at[idx_v.at[0]], bufs[b], gsems[b]).wait()
            pltpu.async_copy(
                bufs[b], out_hbm.at[pl.ds(base + j * CHUNK, CHUNK)], wsems[b]
            )
        for b in range(NBUF):
            pltpu.make_async_copy(
                bufs[b], out_hbm.at[pl.ds(base, CHUNK)], wsems[b]
            ).wait()

            @pl.when(r < ROUNDS - 1)
            def _():
                pltpu.async_copy(
                    table_hbm.at[idx_v.at[(r + 1) * NBUF + b]], bufs[b], gsems[b]
                )

        return 0

    lax.fori_loop(0, ROUNDS, round_body, 0)


def kernel(token_ids, table):
    idx = token_ids.reshape(NUM_WORKERS, NUM_CHUNKS, CHUNK).astype(jnp.int32)
    out = _gather_kernel(idx, table)
    return out.reshape(BATCH, HIST, EMBED_DIM)


# speedup vs baseline: 5.6783x; 1.7129x over previous
import functools

import jax
import jax.numpy as jnp
from jax import lax
from jax.experimental import pallas as pl
from jax.experimental.pallas import tpu as pltpu
from jax.experimental.pallas import tpu_sc as plsc

NUM_EMBEDDINGS = 100000
EMBED_DIM = 128
BATCH = 4096
HIST = 50

NUM_CORES = 2
NUM_SUBCORES = 16
NUM_WORKERS = NUM_CORES * NUM_SUBCORES
ROWS_PER_WORKER = BATCH // NUM_WORKERS
SLAB = 8
NUM_SLABS = ROWS_PER_WORKER // SLAB
NBUF = 2
ROUNDS = NUM_SLABS // NBUF

_mesh = plsc.VectorSubcoreMesh(
    core_axis_name="c",
    subcore_axis_name="s",
    num_cores=NUM_CORES,
    num_subcores=NUM_SUBCORES,
)


@functools.partial(
    pl.kernel,
    out_type=jax.ShapeDtypeStruct((BATCH, HIST, EMBED_DIM), jnp.float32),
    mesh=_mesh,
    scratch_types=[
        pltpu.VMEM((ROWS_PER_WORKER, HIST), jnp.int32),
        [pltpu.VMEM((SLAB, HIST, EMBED_DIM), jnp.float32)] * NBUF,
        [pltpu.SemaphoreType.DMA] * NBUF,
        [pltpu.SemaphoreType.DMA] * NBUF,
    ],
)
def _gather_kernel(idx_hbm, table_hbm, out_hbm, idx_v, slabs, gsems, wsems):
    wid = lax.axis_index("s") * NUM_CORES + lax.axis_index("c")
    wbase = wid * ROWS_PER_WORKER
    pltpu.sync_copy(idx_hbm.at[pl.ds(wbase, ROWS_PER_WORKER)], idx_v)

    def fire_slab(s, p):
        for k in range(SLAB):
            pltpu.async_copy(
                table_hbm.at[idx_v.at[s * SLAB + k]], slabs[p].at[k], gsems[p]
            )

    for p in range(NBUF):
        fire_slab(p, p)

    def round_body(r, _):
        for p in range(NBUF):
            s = r * NBUF + p
            pltpu.make_async_copy(
                out_hbm.at[pl.ds(0, SLAB)], slabs[p], gsems[p]
            ).wait()
            pltpu.async_copy(
                slabs[p], out_hbm.at[pl.ds(wbase + s * SLAB, SLAB)], wsems[p]
            )
        for p in range(NBUF):
            pltpu.make_async_copy(
                slabs[p], out_hbm.at[pl.ds(wbase, SLAB)], wsems[p]
            ).wait()

            @pl.when(r < ROUNDS - 1)
            def _():
                fire_slab((r + 1) * NBUF + p, p)

        return 0

    lax.fori_loop(0, ROUNDS, round_body, 0)


def kernel(token_ids, table):
    return _gather_kernel(token_ids.astype(jnp.int32), table)


# speedup vs baseline: 5.6839x; 1.0010x over previous
import functools

import jax
import jax.numpy as jnp
from jax import lax
from jax.experimental import pallas as pl
from jax.experimental.pallas import tpu as pltpu
from jax.experimental.pallas import tpu_sc as plsc

NUM_EMBEDDINGS = 100000
EMBED_DIM = 128
BATCH = 4096
HIST = 50

NUM_CORES = 2
NUM_SUBCORES = 16
NUM_WORKERS = NUM_CORES * NUM_SUBCORES
ROWS_PER_WORKER = BATCH // NUM_WORKERS
SLAB = 8
NUM_SLABS = ROWS_PER_WORKER // SLAB
NBUF = 2
ROUNDS = NUM_SLABS // NBUF

_mesh = plsc.VectorSubcoreMesh(
    core_axis_name="c",
    subcore_axis_name="s",
    num_cores=NUM_CORES,
    num_subcores=NUM_SUBCORES,
)


@functools.partial(
    pl.kernel,
    out_type=jax.ShapeDtypeStruct((BATCH, HIST, EMBED_DIM), jnp.float32),
    mesh=_mesh,
    compiler_params=pltpu.CompilerParams(use_tc_tiling_on_sc=True),
    scratch_types=[
        pltpu.VMEM((ROWS_PER_WORKER, HIST), jnp.int32),
        [pltpu.VMEM((SLAB, HIST, EMBED_DIM), jnp.float32)] * NBUF,
        [pltpu.SemaphoreType.DMA] * NBUF,
        [pltpu.SemaphoreType.DMA] * NBUF,
    ],
)
def _gather_kernel(idx_hbm, table_hbm, out_hbm, idx_v, slabs, gsems, wsems):
    wid = lax.axis_index("s") * NUM_CORES + lax.axis_index("c")
    wbase = wid * ROWS_PER_WORKER
    pltpu.sync_copy(idx_hbm.at[pl.ds(wbase, ROWS_PER_WORKER)], idx_v)

    def fire_slab(s, p):
        for k in range(SLAB):
            pltpu.async_copy(
                table_hbm.at[idx_v.at[s * SLAB + k]], slabs[p].at[k], gsems[p]
            )

    for p in range(NBUF):
        fire_slab(p, p)

    def round_body(r, _):
        for p in range(NBUF):
            s = r * NBUF + p
            pltpu.make_async_copy(
                out_hbm.at[pl.ds(0, SLAB)], slabs[p], gsems[p]
            ).wait()
            pltpu.async_copy(
                slabs[p], out_hbm.at[pl.ds(wbase + s * SLAB, SLAB)], wsems[p]
            )
        for p in range(NBUF):
            pltpu.make_async_copy(
                slabs[p], out_hbm.at[pl.ds(wbase, SLAB)], wsems[p]
            ).wait()

            @pl.when(r < ROUNDS - 1)
            def _():
                fire_slab((r + 1) * NBUF + p, p)

        return 0

    lax.fori_loop(0, ROUNDS, round_body, 0)


def kernel(token_ids, table):
    return _gather_kernel(token_ids.astype(jnp.int32), table)


# speedup vs baseline: 10.4252x; 1.8342x over previous
import functools

import jax
import jax.numpy as jnp
from jax import lax
from jax.experimental import pallas as pl
from jax.experimental.pallas import tpu as pltpu
from jax.experimental.pallas import tpu_sc as plsc

NUM_EMBEDDINGS = 100000
EMBED_DIM = 128
BATCH = 4096
HIST = 50

NUM_CORES = 2
NUM_SUBCORES = 16
NUM_WORKERS = NUM_CORES * NUM_SUBCORES
BPW = BATCH // NUM_WORKERS
NBUF = 5
ROUNDS = HIST // NBUF

_mesh = plsc.VectorSubcoreMesh(
    core_axis_name="c",
    subcore_axis_name="s",
    num_cores=NUM_CORES,
    num_subcores=NUM_SUBCORES,
)


@functools.partial(
    pl.kernel,
    out_type=jax.ShapeDtypeStruct((HIST, BATCH, EMBED_DIM), jnp.float32),
    mesh=_mesh,
    scratch_types=[
        pltpu.VMEM((HIST, BPW), jnp.int32),
        [pltpu.VMEM((BPW, EMBED_DIM), jnp.float32)] * NBUF,
        [pltpu.SemaphoreType.DMA] * NBUF,
        [pltpu.SemaphoreType.DMA] * NBUF,
    ],
)
def _gather_kernel(idx_hbm, table_hbm, out_hbm, idx_v, bufs, gsems, wsems):
    wid = lax.axis_index("s") * NUM_CORES + lax.axis_index("c")
    wbase = wid * BPW
    pltpu.sync_copy(idx_hbm.at[:, wid], idx_v)

    for b in range(NBUF):
        pltpu.async_copy(table_hbm.at[idx_v.at[b]], bufs[b], gsems[b])

    def round_body(r, _):
        for b in range(NBUF):
            h = r * NBUF + b
            pltpu.make_async_copy(table_hbm.at[idx_v.at[0]], bufs[b], gsems[b]).wait()
            pltpu.async_copy(
                bufs[b], out_hbm.at[h, pl.ds(wbase, BPW)], wsems[b]
            )
        for b in range(NBUF):
            pltpu.make_async_copy(
                bufs[b], out_hbm.at[0, pl.ds(wbase, BPW)], wsems[b]
            ).wait()

            @pl.when(r < ROUNDS - 1)
            def _():
                pltpu.async_copy(
                    table_hbm.at[idx_v.at[(r + 1) * NBUF + b]], bufs[b], gsems[b]
                )

        return 0

    lax.fori_loop(0, ROUNDS, round_body, 0)


def kernel(token_ids, table):
    idx = token_ids.astype(jnp.int32).T.reshape(HIST, NUM_WORKERS, BPW)
    out_hbd = _gather_kernel(idx, table)
    return out_hbd.transpose(1, 0, 2)
